# Initial kernel scaffold; baseline (speedup 1.0000x reference)
import jax
import jax.numpy as jnp
from jax.experimental import pallas as pl


def kernel(pos, params, edge_index, batch):
    raise NotImplementedError("write your pallas kernel here")


# speedup vs baseline: 28.2227x; 28.2227x over previous
import functools

import jax
import jax.numpy as jnp
from jax import lax
from jax.experimental import pallas as pl
from jax.experimental.pallas import tpu as pltpu
from jax.experimental.pallas import tpu_sc as plsc

_NC = 2
_NS = 16
_NW = _NC * _NS
_IW = 128
_BLK = 16
_NEG_SLOPE = 0.01
_NGRAPH = 64
_SC_PARAMS = pltpu.CompilerParams(use_tc_tiling_on_sc=False)


def _leaky(x):
    return jnp.where(x >= 0, x, _NEG_SLOPE * x)


def _sc_gather(table, idx2d):
    R, iw = idx2d.shape
    D = table.shape[1]
    nblk = R // _BLK
    per_w = nblk // _NW
    mesh = plsc.VectorSubcoreMesh(core_axis_name="c", subcore_axis_name="s")

    @functools.partial(
        pl.kernel,
        out_type=jax.ShapeDtypeStruct((R * iw, D), jnp.float32),
        mesh=mesh,
        scratch_types=[
            pltpu.VMEM((_BLK, iw), jnp.int32),
            pltpu.VMEM((_BLK * iw, D), jnp.float32),
            pltpu.SemaphoreType.DMA,
        ],
        compiler_params=_SC_PARAMS,
    )
    def k(table_h, idx_h, out_h, idx_v, rows_v, sem):
        wid = lax.axis_index("s") * _NC + lax.axis_index("c")

        def body(b, carry):
            blk = wid * per_w + b
            r0 = pl.multiple_of(blk * _BLK, _BLK)
            e0 = pl.multiple_of(blk * (_BLK * iw), _BLK * iw)
            pltpu.sync_copy(idx_h.at[pl.ds(r0, _BLK)], idx_v)
            cps = [
                pltpu.async_copy(
                    table_h.at[idx_v.at[j]],
                    rows_v.at[pl.ds(j * iw, iw)],
                    sem,
                )
                for j in range(_BLK)
            ]
            for cp in cps:
                cp.wait()
            pltpu.sync_copy(rows_v, out_h.at[pl.ds(e0, _BLK * iw)])
            return carry

        lax.fori_loop(0, per_w, body, 0)

    return k(table, idx2d)


def _sc_scatter_add(vals, idx2d, acc_rows, zeros_chunk):
    R, iw = idx2d.shape
    D = vals.shape[1]
    blk = 8
    nblk = R // blk
    per_w = nblk // _NW
    slc = acc_rows // _NS
    nch = 8
    ch = slc // nch
    mesh = plsc.VectorSubcoreMesh(core_axis_name="c", subcore_axis_name="s")

    @functools.partial(
        pl.kernel,
        out_type=jax.ShapeDtypeStruct((_NC, acc_rows, D), jnp.float32),
        mesh=mesh,
        scratch_types=[
            pltpu.VMEM((blk, iw), jnp.int32),
            pltpu.VMEM((blk * iw, D), jnp.float32),
            pltpu.VMEM((ch, D), jnp.float32),
            pltpu.VMEM_SHARED((acc_rows, D), jnp.float32),
            pltpu.SemaphoreType.DMA,
        ],
        compiler_params=_SC_PARAMS,
    )
    def k(vals_h, idx_h, zch_h, out_h, idx_v, vals_v, stage_v, acc_sh, sem):
        c = lax.axis_index("c")
        s = lax.axis_index("s")
        wid = s * _NC + c

        pltpu.sync_copy(zch_h, stage_v)

        def zbody(q, carry):
            row0 = pl.multiple_of(s * slc + q * ch, ch)
            pltpu.sync_copy(stage_v, acc_sh.at[pl.ds(row0, ch)])
            return carry

        lax.fori_loop(0, nch, zbody, 0)
        plsc.subcore_barrier()

        def body(b, carry):
            bid = wid * per_w + b
            r0 = pl.multiple_of(bid * blk, blk)
            e0 = pl.multiple_of(bid * (blk * iw), blk * iw)
            pltpu.sync_copy(idx_h.at[pl.ds(r0, blk)], idx_v)
            pltpu.sync_copy(vals_h.at[pl.ds(e0, blk * iw)], vals_v)
            cps = [
                pltpu.async_copy(
                    vals_v.at[pl.ds(j * iw, iw)],
                    acc_sh.at[idx_v.at[j]],
                    sem,
                    add=True,
                )
                for j in range(blk)
            ]
            for cp in cps:
                cp.wait()
            return carry

        lax.fori_loop(0, per_w, body, 0)
        plsc.subcore_barrier()

        def fbody(q, carry):
            row0 = pl.multiple_of(s * slc + q * ch, ch)
            pltpu.sync_copy(acc_sh.at[pl.ds(row0, ch)], stage_v)
            pltpu.sync_copy(stage_v, out_h.at[c, pl.ds(row0, ch)])
            return carry

        lax.fori_loop(0, nch, fbody, 0)

    return k(vals, idx2d, zeros_chunk)


def _graph_sums(pos4, batch2, n_blk):
    N = pos4.shape[0]
    B = N // n_blk

    def body(pos_ref, bat_ref, out_ref):
        @pl.when(pl.program_id(0) == 0)
        def _():
            out_ref[...] = jnp.zeros_like(out_ref)

        onehot = (bat_ref[...] ==
                  lax.broadcasted_iota(jnp.int32, (1, _NGRAPH), 1)
                  ).astype(jnp.float32)
        out_ref[...] += lax.dot_general(
            onehot, pos_ref[...], (((0,), (0,)), ((), ())),
            preferred_element_type=jnp.float32)

    return pl.pallas_call(
        body,
        grid=(n_blk,),
        in_specs=[
            pl.BlockSpec((B, 4), lambda i: (i, 0)),
            pl.BlockSpec((B, 1), lambda i: (i, 0)),
        ],
        out_specs=pl.BlockSpec((_NGRAPH, 4), lambda i: (0, 0)),
        out_shape=jax.ShapeDtypeStruct((_NGRAPH, 4), jnp.float32),
    )(pos4, batch2)


def _center_and_theta(pos4, batch2, sums, n_blk):
    N = pos4.shape[0]
    B = N // n_blk

    def body(pos_ref, bat_ref, sums_ref, posc_ref, rt_ref):
        sums = sums_ref[...]
        cnt = jnp.maximum(sums[:, 3:4], 1.0)
        mean4 = sums / cnt
        onehot = (bat_ref[...] ==
                  lax.broadcasted_iota(jnp.int32, (1, _NGRAPH), 1)
                  ).astype(jnp.float32)
        m = jnp.dot(onehot, mean4, preferred_element_type=jnp.float32)
        pc = pos_ref[...] - m
        posc_ref[...] = jnp.concatenate(
            [pc, jnp.zeros((pc.shape[0], 12), jnp.float32)], axis=1)
        x = pc[:, 0:1]
        y = pc[:, 1:2]
        r = jnp.sqrt(x * x + y * y)
        safe = r > 0
        inv = jnp.where(safe, 1.0 / jnp.where(safe, r, 1.0), 0.0)
        cth = jnp.where(safe, x * inv, 1.0)
        sth = jnp.where(safe, y * inv, 0.0)
        rt_ref[...] = jnp.concatenate([cth, sth], axis=1)

    return pl.pallas_call(
        body,
        grid=(n_blk,),
        in_specs=[
            pl.BlockSpec((B, 4), lambda i: (i, 0)),
            pl.BlockSpec((B, 1), lambda i: (i, 0)),
            pl.BlockSpec((_NGRAPH, 4), lambda i: (0, 0)),
        ],
        out_specs=[
            pl.BlockSpec((B, 16), lambda i: (i, 0)),
            pl.BlockSpec((B, 2), lambda i: (i, 0)),
        ],
        out_shape=[
            jax.ShapeDtypeStruct((N, 16), jnp.float32),
            jax.ShapeDtypeStruct((N, 2), jnp.float32),
        ],
    )(pos4, batch2, sums)


def _edge_geom(pr, pc, n_blk):
    E = pr.shape[0]
    B = E // n_blk

    def body(pr_ref, pc_ref, ecs_ref, demb_ref):
        ev = pr_ref[...] - pc_ref[...]
        x = ev[:, 0:1]
        y = ev[:, 1:2]
        z = ev[:, 2:3]
        rxy2 = x * x + y * y
        d = jnp.sqrt(rxy2 + z * z)
        rxy = jnp.sqrt(rxy2)
        safe = rxy > 0
        inv = jnp.where(safe, 1.0 / jnp.where(safe, rxy, 1.0), 0.0)
        cth = jnp.where(safe, x * inv, 1.0)
        sth = jnp.where(safe, y * inv, 0.0)
        one = jnp.ones_like(d)
        ecs_ref[...] = jnp.concatenate(
            [cth, sth, d, one,
             jnp.zeros((d.shape[0], 12), jnp.float32)], axis=1)
        n = (lax.broadcasted_iota(jnp.int32, (1, 16), 1) + 1
             ).astype(jnp.float32)
        demb_ref[...] = jnp.sin(jnp.pi * n * d) / (d + 1e-9)

    return pl.pallas_call(
        body,
        grid=(n_blk,),
        in_specs=[
            pl.BlockSpec((B, 16), lambda i: (i, 0)),
            pl.BlockSpec((B, 16), lambda i: (i, 0)),
        ],
        out_specs=[
            pl.BlockSpec((B, 16), lambda i: (i, 0)),
            pl.BlockSpec((B, 16), lambda i: (i, 0)),
        ],
        out_shape=[
            jax.ShapeDtypeStruct((E, 16), jnp.float32),
            jax.ShapeDtypeStruct((E, 16), jnp.float32),
        ],
    )(pr, pc)


def _init_nodes(agg4, w_es_row, u0_row, u1_row, w_top0, n_blk, N):
    B = N // n_blk

    def body(agg_ref, wes_ref, u0_ref, u1_ref, wt_ref,
             xs_ref, xr_ref, xsw_ref):
        agg = agg_ref[0] + agg_ref[1]
        cth = agg[:, 0:1]
        sth = agg[:, 1:2]
        dsum = agg[:, 2:3]
        xs = dsum * wes_ref[...]
        xs_ref[...] = xs
        a = cth * u0_ref[...] - sth * u1_ref[...]
        b = sth * u0_ref[...] + cth * u1_ref[...]
        xr_ref[...] = jnp.concatenate([a, b], axis=1)
        xsw_ref[...] = jnp.dot(xs, wt_ref[...],
                               preferred_element_type=jnp.float32)

    return pl.pallas_call(
        body,
        grid=(n_blk,),
        in_specs=[
            pl.BlockSpec((2, B, 16), lambda i: (0, i, 0)),
            pl.BlockSpec((1, 16), lambda i: (0, 0)),
            pl.BlockSpec((1, 16), lambda i: (0, 0)),
            pl.BlockSpec((1, 16), lambda i: (0, 0)),
            pl.BlockSpec((16, 16), lambda i: (0, 0)),
        ],
        out_specs=[
            pl.BlockSpec((B, 16), lambda i: (i, 0)),
            pl.BlockSpec((B, 32), lambda i: (i, 0)),
            pl.BlockSpec((B, 16), lambda i: (i, 0)),
        ],
        out_shape=[
            jax.ShapeDtypeStruct((N, 16), jnp.float32),
            jax.ShapeDtypeStruct((N, 32), jnp.float32),
            jax.ShapeDtypeStruct((N, 16), jnp.float32),
        ],
    )(agg4, w_es_row, u0_row, u1_row, w_top0)


def _edge_msgs(sin_e, rin, demb, ecs, w_bot, b_row, w_gate, n_blk):
    E = sin_e.shape[0]
    B = E // n_blk

    def body(sin_ref, rin_ref, demb_ref, ecs_ref, wb_ref, b_ref, wg_ref,
             ms_ref, r0_ref, r1_ref):
        demb = demb_ref[...]
        pe = jnp.dot(demb, wb_ref[...], preferred_element_type=jnp.float32)
        ms_ref[...] = _leaky(sin_ref[...] + pe + b_ref[...])
        g = jnp.dot(demb, wg_ref[...], preferred_element_type=jnp.float32)
        rin = rin_ref[...]
        a = rin[:, 0:16]
        b = rin[:, 16:32]
        cth = ecs_ref[:, 0:1]
        sth = ecs_ref[:, 1:2]
        r0_ref[...] = g * (a * cth + b * sth)
        r1_ref[...] = g * (b * cth - a * sth)

    return pl.pallas_call(
        body,
        grid=(n_blk,),
        in_specs=[
            pl.BlockSpec((B, 16), lambda i: (i, 0)),
            pl.BlockSpec((B, 32), lambda i: (i, 0)),
            pl.BlockSpec((B, 16), lambda i: (i, 0)),
            pl.BlockSpec((B, 16), lambda i: (i, 0)),
            pl.BlockSpec((16, 16), lambda i: (0, 0)),
            pl.BlockSpec((1, 16), lambda i: (0, 0)),
            pl.BlockSpec((16, 16), lambda i: (0, 0)),
        ],
        out_specs=[
            pl.BlockSpec((B, 16), lambda i: (i, 0)),
            pl.BlockSpec((B, 16), lambda i: (i, 0)),
            pl.BlockSpec((B, 16), lambda i: (i, 0)),
        ],
        out_shape=[
            jax.ShapeDtypeStruct((E, 16), jnp.float32),
            jax.ShapeDtypeStruct((E, 16), jnp.float32),
            jax.ShapeDtypeStruct((E, 16), jnp.float32),
        ],
    )(sin_e, rin, demb, ecs, w_bot, b_row, w_gate)


def _update_nodes(xs, xr, aggs, aggr0, aggr1, w_upd_s, w_upd_r, w_top_next,
                  n_blk):
    N = xs.shape[0]
    B = N // n_blk

    def body(xs_ref, xr_ref, as_ref, ar0_ref, ar1_ref, wus_ref, wur_ref,
             wt_ref, xs_o, xr_o, xsw_o):
        aggs = as_ref[0] + as_ref[1]
        ar0 = ar0_ref[0] + ar0_ref[1]
        ar1 = ar1_ref[0] + ar1_ref[1]
        xs = xs_ref[...] + jnp.dot(aggs, wus_ref[...],
                                   preferred_element_type=jnp.float32)
        xs_o[...] = xs
        xr = xr_ref[...]
        a = xr[:, 0:16] + jnp.dot(ar0, wur_ref[...],
                                  preferred_element_type=jnp.float32)
        b = xr[:, 16:32] + jnp.dot(ar1, wur_ref[...],
                                   preferred_element_type=jnp.float32)
        xr_o[...] = jnp.concatenate([a, b], axis=1)
        xsw_o[...] = jnp.dot(xs, wt_ref[...],
                             preferred_element_type=jnp.float32)

    return pl.pallas_call(
        body,
        grid=(n_blk,),
        in_specs=[
            pl.BlockSpec((B, 16), lambda i: (i, 0)),
            pl.BlockSpec((B, 32), lambda i: (i, 0)),
            pl.BlockSpec((2, B, 16), lambda i: (0, i, 0)),
            pl.BlockSpec((2, B, 16), lambda i: (0, i, 0)),
            pl.BlockSpec((2, B, 16), lambda i: (0, i, 0)),
            pl.BlockSpec((16, 16), lambda i: (0, 0)),
            pl.BlockSpec((16, 16), lambda i: (0, 0)),
            pl.BlockSpec((16, 16), lambda i: (0, 0)),
        ],
        out_specs=[
            pl.BlockSpec((B, 16), lambda i: (i, 0)),
            pl.BlockSpec((B, 32), lambda i: (i, 0)),
            pl.BlockSpec((B, 16), lambda i: (i, 0)),
        ],
        out_shape=[
            jax.ShapeDtypeStruct((N, 16), jnp.float32),
            jax.ShapeDtypeStruct((N, 32), jnp.float32),
            jax.ShapeDtypeStruct((N, 16), jnp.float32),
        ],
    )(xs, xr, aggs, aggr0, aggr1, w_upd_s, w_upd_r, w_top_next)


def _readout(xs, xr, rt, batch2, w1, b1_row, w2, b2_row, n_blk):
    N = xs.shape[0]
    B = N // n_blk

    def body(xs_ref, xr_ref, rt_ref, bat_ref, w1_ref, b1_ref, w2_ref,
             b2_ref, out_ref):
        @pl.when(pl.program_id(0) == 0)
        def _():
            out_ref[...] = jnp.zeros_like(out_ref)

        xr = xr_ref[...]
        a = xr[:, 0:16]
        b = xr[:, 16:32]
        cth = rt_ref[:, 0:1]
        sth = rt_ref[:, 1:2]
        o0 = a * cth - b * sth
        o1 = a * sth + b * cth
        inter = jnp.stack([o0, o1], axis=-1).reshape(o0.shape[0], 32)
        h = jnp.concatenate([xs_ref[...], inter], axis=1)
        z = _leaky(jnp.dot(h, w1_ref[...],
                           preferred_element_type=jnp.float32) + b1_ref[...])
        u = jnp.dot(z, w2_ref[...],
                    preferred_element_type=jnp.float32) + b2_ref[...]
        onehot = (bat_ref[...] ==
                  lax.broadcasted_iota(jnp.int32, (1, _NGRAPH), 1)
                  ).astype(jnp.float32)
        out_ref[...] += lax.dot_general(
            onehot, u, (((0,), (0,)), ((), ())),
            preferred_element_type=jnp.float32)

    return pl.pallas_call(
        body,
        grid=(n_blk,),
        in_specs=[
            pl.BlockSpec((B, 16), lambda i: (i, 0)),
            pl.BlockSpec((B, 32), lambda i: (i, 0)),
            pl.BlockSpec((B, 2), lambda i: (i, 0)),
            pl.BlockSpec((B, 1), lambda i: (i, 0)),
            pl.BlockSpec((48, 144), lambda i: (0, 0)),
            pl.BlockSpec((1, 144), lambda i: (0, 0)),
            pl.BlockSpec((144, 6), lambda i: (0, 0)),
            pl.BlockSpec((1, 6), lambda i: (0, 0)),
        ],
        out_specs=pl.BlockSpec((_NGRAPH, 6), lambda i: (0, 0)),
        out_shape=jax.ShapeDtypeStruct((_NGRAPH, 6), jnp.float32),
    )(xs, xr, rt, batch2, w1, b1_row, w2, b2_row)


def kernel(pos, params, edge_index, batch):
    N = pos.shape[0]
    E = edge_index.shape[1]

    grp = _IW * _BLK * _NW
    e_pad = ((E + grp - 1) // grp) * grp
    acc_rows = ((N + 1 + 127) // 128) * 128

    pos4 = jnp.concatenate(
        [pos, jnp.ones((N, 1), jnp.float32)], axis=1)
    batch2 = batch[:, None]
    row = edge_index[0]
    col = edge_index[1]
    padg = jnp.zeros((e_pad - E,), jnp.int32)
    row_g = jnp.concatenate([row, padg]).reshape(e_pad // _IW, _IW)
    col_g = jnp.concatenate([col, padg]).reshape(e_pad // _IW, _IW)
    pads = jnp.full((e_pad - E,), N, jnp.int32)
    col_s = jnp.concatenate([col, pads]).reshape(e_pad // _IW, _IW)

    zch16 = jnp.zeros((acc_rows // _NS // 8, 16), jnp.float32)

    w_es_row = params['w_es'][None, :]
    u0_row = params['u_emb'][:, 0][None, :]
    u1_row = params['u_emb'][:, 1][None, :]
    w_top = [params['W_msg_s_%d' % i][:16] for i in range(3)]
    w_bot = [params['W_msg_s_%d' % i][16:] for i in range(3)]
    b_msg = [params['b_msg_s_%d' % i][None, :] for i in range(3)]
    w_gate = [params['W_gate_%d' % i] for i in range(3)]
    w_upd_s = [params['W_upd_s_%d' % i] for i in range(3)]
    w_upd_r = [params['W_upd_r_%d' % i] for i in range(3)]
    b1_row = params['b1'][None, :]
    b2_row = params['b2'][None, :]

    n_blk_n = 50
    n_blk_e = e_pad // 4096

    sums = _graph_sums(pos4, batch2, n_blk_n)
    posc16, rt = _center_and_theta(pos4, batch2, sums, n_blk_n)
    pr = _sc_gather(posc16, row_g)
    pc = _sc_gather(posc16, col_g)
    ecs, demb = _edge_geom(pr, pc, n_blk_e)

    agg4 = _sc_scatter_add(ecs, col_s, acc_rows, zch16)
    agg4 = agg4[:, :N, :]
    xs, xr, xsw = _init_nodes(agg4, w_es_row, u0_row, u1_row, w_top[0],
                              n_blk_n, N)

    for i in range(3):
        sin_e = _sc_gather(xsw, row_g)
        rin = _sc_gather(xr, row_g)
        ms, r0, r1 = _edge_msgs(sin_e, rin, demb, ecs, w_bot[i], b_msg[i],
                                w_gate[i], n_blk_e)
        aggs = _sc_scatter_add(ms, col_s, acc_rows, zch16)[:, :N, :]
        aggr0 = _sc_scatter_add(r0, col_s, acc_rows, zch16)[:, :N, :]
        aggr1 = _sc_scatter_add(r1, col_s, acc_rows, zch16)[:, :N, :]
        w_next = w_top[i + 1] if i < 2 else w_upd_s[i]
        xs, xr, xsw = _update_nodes(xs, xr, aggs, aggr0, aggr1, w_upd_s[i],
                                    w_upd_r[i], w_next, n_blk_n)

    return _readout(xs, xr, rt, batch2, params['W1'], b1_row,
                    params['W2'], b2_row, n_blk_n)
